# baseline (speedup 1.0000x reference)
import jax
import jax.numpy as jnp
from jax.experimental import pallas as pl
from jax.experimental.pallas import tpu as pltpu

_R = 400


def _oh_kernel(k_ref, sp_ref, p_ref, cls_ref, out_ref):
    k0 = k_ref[0]
    k1 = k_ref[1]
    sp = sp_ref[...]
    hi = sp >= 10
    base = jnp.where(hi, sp - 10, sp)
    kv = jnp.where(hi, k1, k0)
    idx = jax.lax.shift_right_logical(kv, base * 3) & 7
    g = jnp.dot(idx.astype(jnp.bfloat16), p_ref[...],
                preferred_element_type=jnp.float32)
    out_ref[...] = (g == cls_ref[...]).astype(jnp.int32)


def kernel(species, species_to_index, n_species):
    n = species.shape[0]
    rows = n // 128
    sp2 = species.reshape(rows, 128)
    s2i = species_to_index.astype(jnp.int32)
    k0 = jnp.sum(s2i[:10] << (jnp.arange(10, dtype=jnp.int32) * 3)).astype(jnp.int32)
    k1 = jnp.sum(s2i[10:] << (jnp.arange(7, dtype=jnp.int32) * 3)).astype(jnp.int32)
    kparams = jnp.stack([k0, k1])

    a = jnp.arange(128, dtype=jnp.int32)
    c = jnp.arange(640, dtype=jnp.int32)
    p = (a[:, None] == (c[None, :] // 5)).astype(jnp.bfloat16)
    cc = c % 5
    cls = jnp.where(cc < n_species, cc, -1).astype(jnp.float32)[None, :]

    grid = rows // _R
    out = pl.pallas_call(
        _oh_kernel,
        grid=(grid,),
        in_specs=[
            pl.BlockSpec(memory_space=pltpu.SMEM),
            pl.BlockSpec((_R, 128), lambda i: (i, 0)),
            pl.BlockSpec((128, 640), lambda i: (0, 0)),
            pl.BlockSpec((1, 640), lambda i: (0, 0)),
        ],
        out_specs=pl.BlockSpec((_R, 640), lambda i: (i, 0)),
        out_shape=jax.ShapeDtypeStruct((rows, 640), jnp.int32),
    )(kparams, sp2, p, cls)
    return out.reshape(n, 5)


# speedup vs baseline: 27.6880x; 27.6880x over previous
import jax
import jax.numpy as jnp
from jax.experimental import pallas as pl
from jax.experimental.pallas import tpu as pltpu

_C = 6400


def _ohT_kernel(k_ref, sp_ref, out_ref):
    k0 = k_ref[0]
    k1 = k_ref[1]
    vm = k_ref[2]
    sp = sp_ref[0]
    hi = sp >= 10
    base = jnp.where(hi, sp - 10, sp)
    kv = jnp.where(hi, k1, k0)
    idx = jax.lax.shift_right_logical(kv, base * 3) & 7
    v = (jnp.int32(1) << idx) & vm
    j5 = jax.lax.broadcasted_iota(jnp.int32, (5, _C), 0)
    for r in range(8):
        row = jnp.broadcast_to(v[r : r + 1, :], (5, _C))
        out_ref[:, r * _C : (r + 1) * _C] = (
            jax.lax.shift_right_logical(row, j5) & 1
        )


def kernel(species, species_to_index, n_species):
    n = species.shape[0]
    cols = 8 * _C
    nb = n // cols
    sp3 = species.reshape(nb, 8, _C)
    s2i = species_to_index.astype(jnp.int32)
    k0 = jnp.sum(s2i[:10] << (jnp.arange(10, dtype=jnp.int32) * 3)).astype(jnp.int32)
    k1 = jnp.sum(s2i[10:] << (jnp.arange(7, dtype=jnp.int32) * 3)).astype(jnp.int32)
    vm = (jnp.int32(1) << jnp.asarray(n_species, jnp.int32)) - 1
    kparams = jnp.stack([k0, k1, vm])

    out_t = pl.pallas_call(
        _ohT_kernel,
        grid=(nb,),
        in_specs=[
            pl.BlockSpec(memory_space=pltpu.SMEM),
            pl.BlockSpec((1, 8, _C), lambda i: (i, 0, 0)),
        ],
        out_specs=pl.BlockSpec((5, cols), lambda i: (0, i)),
        out_shape=jax.ShapeDtypeStruct((5, n), jnp.int32),
    )(kparams, sp3)
    return out_t.T


# speedup vs baseline: 42.0888x; 1.5201x over previous
import jax
import jax.numpy as jnp
from jax.experimental import pallas as pl
from jax.experimental.pallas import tpu as pltpu

_C = 32000


def _ohT_kernel(k_ref, sp_ref, out_ref):
    k0 = k_ref[0]
    k1 = k_ref[1]
    vm = k_ref[2]
    sp = sp_ref[0]
    hi = sp >= 10
    base = jnp.where(hi, sp - 10, sp)
    kv = jnp.where(hi, k1, k0)
    idx = jax.lax.shift_right_logical(kv, base * 3) & 7
    v = (jnp.int32(1) << idx) & vm
    j5 = jax.lax.broadcasted_iota(jnp.int32, (5, _C), 0)
    for r in range(8):
        row = jnp.broadcast_to(v[r : r + 1, :], (5, _C))
        out_ref[:, r * _C : (r + 1) * _C] = (
            jax.lax.shift_right_logical(row, j5) & 1
        )


def kernel(species, species_to_index, n_species):
    n = species.shape[0]
    cols = 8 * _C
    nb = n // cols
    sp3 = species.reshape(nb, 8, _C)
    s2i = species_to_index.astype(jnp.int32)
    k0 = jnp.sum(s2i[:10] << (jnp.arange(10, dtype=jnp.int32) * 3)).astype(jnp.int32)
    k1 = jnp.sum(s2i[10:] << (jnp.arange(7, dtype=jnp.int32) * 3)).astype(jnp.int32)
    vm = (jnp.int32(1) << jnp.asarray(n_species, jnp.int32)) - 1
    kparams = jnp.stack([k0, k1, vm])

    out_t = pl.pallas_call(
        _ohT_kernel,
        grid=(nb,),
        in_specs=[
            pl.BlockSpec(memory_space=pltpu.SMEM),
            pl.BlockSpec((1, 8, _C), lambda i: (i, 0, 0)),
        ],
        out_specs=pl.BlockSpec((5, cols), lambda i: (0, i)),
        out_shape=jax.ShapeDtypeStruct((5, n), jnp.int32),
    )(kparams, sp3)
    return out_t.T


# speedup vs baseline: 42.6419x; 1.0131x over previous
import jax
import jax.numpy as jnp
from jax.experimental import pallas as pl
from jax.experimental.pallas import tpu as pltpu

_C = 80000


def _ohT_kernel(k_ref, sp_ref, out_ref):
    k0 = k_ref[0]
    k1 = k_ref[1]
    vm = k_ref[2]
    sp = sp_ref[0]
    hi = sp >= 10
    base = jnp.where(hi, sp - 10, sp)
    kv = jnp.where(hi, k1, k0)
    idx = jax.lax.shift_right_logical(kv, base * 3) & 7
    v = (jnp.int32(1) << idx) & vm
    j5 = jax.lax.broadcasted_iota(jnp.int32, (5, _C), 0)
    for r in range(8):
        row = jnp.broadcast_to(v[r : r + 1, :], (5, _C))
        out_ref[:, r * _C : (r + 1) * _C] = (
            jax.lax.shift_right_logical(row, j5) & 1
        )


def kernel(species, species_to_index, n_species):
    n = species.shape[0]
    cols = 8 * _C
    nb = n // cols
    sp3 = species.reshape(nb, 8, _C)
    s2i = species_to_index.astype(jnp.int32)
    k0 = jnp.sum(s2i[:10] << (jnp.arange(10, dtype=jnp.int32) * 3)).astype(jnp.int32)
    k1 = jnp.sum(s2i[10:] << (jnp.arange(7, dtype=jnp.int32) * 3)).astype(jnp.int32)
    vm = (jnp.int32(1) << jnp.asarray(n_species, jnp.int32)) - 1
    kparams = jnp.stack([k0, k1, vm])

    out_t = pl.pallas_call(
        _ohT_kernel,
        grid=(nb,),
        in_specs=[
            pl.BlockSpec(memory_space=pltpu.SMEM),
            pl.BlockSpec((1, 8, _C), lambda i: (i, 0, 0)),
        ],
        out_specs=pl.BlockSpec((5, cols), lambda i: (0, i)),
        out_shape=jax.ShapeDtypeStruct((5, n), jnp.int32),
    )(kparams, sp3)
    return out_t.T
